# baseline (speedup 1.0000x reference)
import functools

import jax
import jax.numpy as jnp
from jax import lax
from jax.experimental import pallas as pl
from jax.experimental.pallas import tpu as pltpu
from jax.experimental.pallas import tpu_sc as plsc

T = 128
LANES = 16


def _gating_kernel(x_ref, gW1_ref, gb1_ref, gln_g_ref, gln_b_ref, gW2_ref,
                   gb2_ref, probs_ref, eid_ref, wgt_ref, xbf_ref, *, E):
    x = x_ref[...]
    xbf_ref[...] = x.astype(jnp.bfloat16)
    B = x.shape[0]
    h = jnp.dot(x, gW1_ref[...], preferred_element_type=jnp.float32)
    h = h + gb1_ref[0]
    mu = jnp.mean(h, axis=-1, keepdims=True)
    var = jnp.mean((h - mu) ** 2, axis=-1, keepdims=True)
    h = (h - mu) * jax.lax.rsqrt(var + 1e-5) * gln_g_ref[0] + gln_b_ref[0]
    h = jax.nn.gelu(h)
    glog = jnp.dot(h, gW2_ref[...], preferred_element_type=jnp.float32)
    glog = glog + gb2_ref[0]
    probs = jax.nn.softmax(glog, axis=-1)
    probs_ref[...] = probs

    e_iota = jax.lax.broadcasted_iota(jnp.int32, (B, E), 1)
    v1 = jnp.max(probs, axis=-1, keepdims=True)
    i1 = jnp.min(jnp.where(probs == v1, e_iota, E), axis=-1, keepdims=True)
    mask1 = e_iota == i1
    probs_rest = jnp.where(mask1, -1.0, probs)
    v2 = jnp.max(probs_rest, axis=-1, keepdims=True)
    i2 = jnp.min(jnp.where(probs_rest == v2, e_iota, E), axis=-1,
                 keepdims=True)
    mask2 = e_iota == i2
    eid_ref[...] = jnp.concatenate([i1, i2], axis=1)
    wgt_ref[...] = jnp.concatenate([v1, v2], axis=1)


def _routing_kernel(eidk_ref, wgtk_ref, pos_ref, tsrc_ref, wpad_ref,
                    tile_ref, *, A, B, E, NPAD):
    eidk = eidk_ref[...]
    e_row = jax.lax.broadcasted_iota(jnp.int32, (A, E), 1)
    oh = (eidk == e_row).astype(jnp.float32)

    CH = 512
    r_lt = (jax.lax.broadcasted_iota(jnp.int32, (CH, CH), 0)
            > jax.lax.broadcasted_iota(jnp.int32, (CH, CH), 1)
            ).astype(jnp.float32)
    carry = jnp.zeros((1, E), jnp.float32)
    rank_parts = []
    for c in range(A // CH):
        oh_c = oh[c * CH:(c + 1) * CH]
        rk_c = jnp.dot(r_lt, oh_c, preferred_element_type=jnp.float32) + carry
        rank_parts.append(jnp.sum(rk_c * oh_c, axis=1, keepdims=True))
        carry = carry + jnp.sum(oh_c, axis=0, keepdims=True)
    rank = jnp.concatenate(rank_parts, axis=0)

    totals = carry.astype(jnp.int32)
    pad = ((totals + (T - 1)) >> 7) << 7
    u_lt = (jax.lax.broadcasted_iota(jnp.int32, (E, E), 0)
            < jax.lax.broadcasted_iota(jnp.int32, (E, E), 1)
            ).astype(jnp.float32)
    gstart = jnp.dot(pad.astype(jnp.float32), u_lt,
                     preferred_element_type=jnp.float32)
    gsel = jnp.sum(oh * gstart, axis=1, keepdims=True)
    pos = (gsel + rank).astype(jnp.int32)
    pos_ref[...] = pos

    a_iota = jax.lax.broadcasted_iota(jnp.int32, (1, A), 1)
    tok_row = jnp.where(a_iota >= B, a_iota - B, a_iota).astype(jnp.float32)
    wgt_row = wgtk_ref[...]
    for c in range(NPAD // CH):
        cols = jax.lax.broadcasted_iota(jnp.int32, (1, CH), 1) + c * CH
        po = (pos == cols).astype(jnp.float32)
        tsrc_ref[:, c * CH:(c + 1) * CH] = jnp.dot(
            tok_row, po, preferred_element_type=jnp.float32).astype(jnp.int32)
        wpad_ref[:, c * CH:(c + 1) * CH] = jnp.dot(
            wgt_row, po, preferred_element_type=jnp.float32)

    n_tile_pad = tile_ref.shape[1]
    tcol = jax.lax.broadcasted_iota(jnp.int32, (1, n_tile_pad), 1) * T
    te = jnp.zeros((1, n_tile_pad), jnp.int32)
    for e in range(E):
        gs = gstart[0, e].astype(jnp.int32)
        inr = (tcol >= gs) & (tcol < gs + pad[0, e])
        te = jnp.where(inr, e, te)
    tile_ref[...] = te


def _make_gather_kernel(NPAD, H2, n_sub, rows_sub):
    mesh = plsc.VectorSubcoreMesh(core_axis_name="c", subcore_axis_name="s")

    @functools.partial(
        pl.kernel, mesh=mesh,
        out_type=jax.ShapeDtypeStruct((NPAD, H2), jnp.int32),
        scratch_types=[
            pltpu.VMEM((n_sub, rows_sub), jnp.int32),
        ] + [pltpu.VMEM((rows_sub, H2), jnp.int32) for _ in range(4)] + [
            pltpu.SemaphoreType.DMA,
            pltpu.SemaphoreType.DMA,
        ],
    )
    def gather(tsrc_hbm, x_hbm, xg_hbm, idxv, b0, b1, b2, b3, semg, sems):
        wid = lax.axis_index("s") * 2 + lax.axis_index("c")
        idx_row0 = pl.multiple_of(wid * n_sub, n_sub)
        base = pl.multiple_of(wid * n_sub * rows_sub, n_sub * rows_sub)
        pltpu.sync_copy(tsrc_hbm.at[pl.ds(idx_row0, n_sub)], idxv)
        bufs = [b0, b1, b2, b3]
        gets = [pltpu.async_copy(x_hbm.at[idxv.at[h]], bufs[h], semg)
                for h in range(n_sub)]
        puts = []
        for h in range(n_sub):
            gets[h].wait()
            puts.append(pltpu.async_copy(
                bufs[h], xg_hbm.at[pl.ds(base + h * rows_sub, rows_sub)],
                sems))
        for p in puts:
            p.wait()

    return gather


def _make_combine_kernel(B, H2, NPAD):
    mesh = plsc.VectorSubcoreMesh(core_axis_name="c", subcore_axis_name="s")
    t_chunk = B // 32
    n_sub = 2
    rows_sub = t_chunk // n_sub

    @functools.partial(
        pl.kernel, mesh=mesh,
        out_type=[
            jax.ShapeDtypeStruct((B, H2), jnp.int32),
            jax.ShapeDtypeStruct((B, H2), jnp.int32),
        ],
        scratch_types=[
            pltpu.VMEM((n_sub, rows_sub), jnp.int32),
            pltpu.VMEM((n_sub, rows_sub), jnp.int32),
        ] + [pltpu.VMEM((rows_sub, H2), jnp.int32) for _ in range(4)] + [
            pltpu.SemaphoreType.DMA,
            pltpu.SemaphoreType.DMA,
        ],
    )
    def combine(y_hbm, pos_hbm, ya_hbm, yb_hbm, idxa, idxb, b0, b1, b2, b3,
                semg, sems):
        wid = lax.axis_index("s") * 2 + lax.axis_index("c")
        arow = pl.multiple_of(wid * n_sub, n_sub)
        brow = pl.multiple_of((B // rows_sub) + wid * n_sub, n_sub)
        pltpu.sync_copy(pos_hbm.at[pl.ds(arow, n_sub)], idxa)
        pltpu.sync_copy(pos_hbm.at[pl.ds(brow, n_sub)], idxb)
        tbase = pl.multiple_of(wid * t_chunk, t_chunk)
        bufs = [b0, b1, b2, b3]
        seq = [(idxa, 0, ya_hbm, 0), (idxa, 1, ya_hbm, 1),
               (idxb, 0, yb_hbm, 0), (idxb, 1, yb_hbm, 1)]
        gets = [pltpu.async_copy(y_hbm.at[ir.at[h]], bufs[t], semg)
                for t, (ir, h, _, _) in enumerate(seq)]
        puts = []
        for t, (ir, h, dst, off) in enumerate(seq):
            gets[t].wait()
            puts.append(pltpu.async_copy(
                bufs[t], dst.at[pl.ds(tbase + off * rows_sub, rows_sub)],
                sems))
        for p in puts:
            p.wait()

    return combine


def _matmul_kernel(te_ref, xg_ref, eW_ref, eb_ref, wgt_ref, y_ref):
    xg = xg_ref[...].astype(jnp.float32)
    y = jnp.dot(xg, eW_ref[0], preferred_element_type=jnp.float32)
    y_ref[...] = (wgt_ref[0] * (y + eb_ref[0])).astype(jnp.bfloat16)


def _classifier_kernel(ya_ref, yb_ref, cW1_ref, cb1_ref, cln_g_ref,
                       cln_b_ref, cW2_ref, cb2_ref, logits_ref):
    ci = (ya_ref[...].astype(jnp.float32)
          + yb_ref[...].astype(jnp.float32))
    ch = jnp.dot(ci, cW1_ref[...], preferred_element_type=jnp.float32)
    ch = ch + cb1_ref[0]
    mu = jnp.mean(ch, axis=-1, keepdims=True)
    var = jnp.mean((ch - mu) ** 2, axis=-1, keepdims=True)
    ch = (ch - mu) * jax.lax.rsqrt(var + 1e-5) * cln_g_ref[0] + cln_b_ref[0]
    ch = jnp.maximum(ch, 0.0)
    logits = jnp.dot(ch, cW2_ref[...], preferred_element_type=jnp.float32)
    logits_ref[...] = logits + cb2_ref[0]


def kernel(x, gW1, gb1, gln_g, gln_b, gW2, gb2, eW, eb, cW1, cb1, cln_g,
           cln_b, cW2, cb2):
    B, H = x.shape
    E = eW.shape[0]
    C = cW2.shape[1]
    A = 2 * B
    NPAD = A + E * T
    NT = NPAD // T
    NTPAD = ((NT + LANES - 1) // LANES) * LANES

    def row(v):
        return v.reshape(1, -1)

    full = lambda a: pl.BlockSpec(a.shape, lambda i: (0,) * a.ndim)

    probs, eid, wgt, xbf = pl.pallas_call(
        functools.partial(_gating_kernel, E=E),
        grid=(1,),
        in_specs=[full(x), full(gW1), full(row(gb1)), full(row(gln_g)),
                  full(row(gln_b)), full(gW2), full(row(gb2))],
        out_specs=[pl.BlockSpec((B, E), lambda i: (0, 0)),
                   pl.BlockSpec((B, 2), lambda i: (0, 0)),
                   pl.BlockSpec((B, 2), lambda i: (0, 0)),
                   pl.BlockSpec((B, H), lambda i: (0, 0))],
        out_shape=[jax.ShapeDtypeStruct((B, E), jnp.float32),
                   jax.ShapeDtypeStruct((B, 2), jnp.int32),
                   jax.ShapeDtypeStruct((B, 2), jnp.float32),
                   jax.ShapeDtypeStruct((B, H), jnp.bfloat16)],
    )(x, gW1, row(gb1), row(gln_g), row(gln_b), gW2, row(gb2))

    pos, t_src, wgt_pad, tile_e = pl.pallas_call(
        functools.partial(_routing_kernel, A=A, B=B, E=E, NPAD=NPAD),
        grid=(1,),
        in_specs=[pl.BlockSpec((A, 1), lambda i: (0, 0)),
                  pl.BlockSpec((1, A), lambda i: (0, 0))],
        out_specs=[pl.BlockSpec((A, 1), lambda i: (0, 0)),
                   pl.BlockSpec((1, NPAD), lambda i: (0, 0)),
                   pl.BlockSpec((1, NPAD), lambda i: (0, 0)),
                   pl.BlockSpec((1, 64), lambda i: (0, 0))],
        out_shape=[jax.ShapeDtypeStruct((A, 1), jnp.int32),
                   jax.ShapeDtypeStruct((1, NPAD), jnp.int32),
                   jax.ShapeDtypeStruct((1, NPAD), jnp.float32),
                   jax.ShapeDtypeStruct((1, 64), jnp.int32)],
    )(eid.T.reshape(A, 1), wgt.T.reshape(1, A))
    t_src = t_src.reshape(NPAD)
    wgt_pad = wgt_pad.reshape(NPAD)
    pos = pos.reshape(A // 64, 64)
    tile_e = tile_e.reshape(64)

    n_sub, rows_sub = 4, NPAD // 32 // 4
    gather = _make_gather_kernel(NPAD, H // 2, n_sub, rows_sub)
    x_i32 = jax.lax.bitcast_convert_type(
        xbf.reshape(B, H // 2, 2), jnp.int32)
    xg_i32 = gather(t_src.reshape(32 * n_sub, rows_sub), x_i32)
    xg = jax.lax.bitcast_convert_type(
        xg_i32.reshape(NPAD, H // 2, 1), jnp.bfloat16).reshape(NPAD, H)

    y = pl.pallas_call(
        _matmul_kernel,
        grid_spec=pltpu.PrefetchScalarGridSpec(
            num_scalar_prefetch=1,
            grid=(NT,),
            in_specs=[
                pl.BlockSpec((T, H), lambda i, te: (i, 0)),
                pl.BlockSpec((1, H, H), lambda i, te: (te[i], 0, 0)),
                pl.BlockSpec((1, 1, H), lambda i, te: (te[i], 0, 0)),
                pl.BlockSpec((1, T, 1), lambda i, te: (i, 0, 0)),
            ],
            out_specs=pl.BlockSpec((T, H), lambda i, te: (i, 0)),
        ),
        out_shape=jax.ShapeDtypeStruct((NPAD, H), jnp.bfloat16),
    )(tile_e[:NT], xg, eW, eb.reshape(E, 1, H), wgt_pad.reshape(NT, T, 1))

    combine = _make_combine_kernel(B, H // 2, NPAD)
    y_i32 = jax.lax.bitcast_convert_type(
        y.reshape(NPAD, H // 2, 2), jnp.int32)
    ya_i, yb_i = combine(y_i32, pos.reshape(A // 32, 32))
    ya = jax.lax.bitcast_convert_type(
        ya_i.reshape(B, H // 2, 1), jnp.bfloat16).reshape(B, H)
    yb = jax.lax.bitcast_convert_type(
        yb_i.reshape(B, H // 2, 1), jnp.bfloat16).reshape(B, H)

    logits = pl.pallas_call(
        _classifier_kernel,
        grid=(1,),
        in_specs=[full(ya), full(yb), full(cW1), full(row(cb1)),
                  full(row(cln_g)), full(row(cln_b)), full(cW2),
                  full(row(cb2))],
        out_specs=pl.BlockSpec((B, C), lambda i: (0, 0)),
        out_shape=jax.ShapeDtypeStruct((B, C), jnp.float32),
    )(ya, yb, cW1, row(cb1), row(cln_g), row(cln_b), cW2, row(cb2))

    return logits, probs


# speedup vs baseline: 7.8153x; 7.8153x over previous
import functools

import jax
import jax.numpy as jnp
from jax.experimental import pallas as pl
from jax.experimental.pallas import tpu as pltpu


def _fused_kernel(x_ref, gW1_ref, gb1_ref, gln_g_ref, gln_b_ref, gW2_ref,
                  gb2_ref, eW_ref, eb_ref, cW1_ref, cb1_ref, cln_g_ref,
                  cln_b_ref, cW2_ref, cb2_ref, logits_ref, probs_ref,
                  acc_ref, comb_ref, *, E):
    e = pl.program_id(0)
    B = x_ref.shape[0]

    @pl.when(e == 0)
    def _gating():
        x = x_ref[...]
        h = jnp.dot(x, gW1_ref[...], preferred_element_type=jnp.float32)
        h = h + gb1_ref[0]
        mu = jnp.mean(h, axis=-1, keepdims=True)
        var = jnp.mean((h - mu) ** 2, axis=-1, keepdims=True)
        h = (h - mu) * jax.lax.rsqrt(var + 1e-5) * gln_g_ref[0] + gln_b_ref[0]
        h = jax.nn.gelu(h)
        glog = jnp.dot(h, gW2_ref[...], preferred_element_type=jnp.float32)
        glog = glog + gb2_ref[0]
        probs = jax.nn.softmax(glog, axis=-1)
        probs_ref[...] = probs

        e_iota = jax.lax.broadcasted_iota(jnp.int32, (B, E), 1)
        v1 = jnp.max(probs, axis=-1, keepdims=True)
        i1 = jnp.min(jnp.where(probs == v1, e_iota, E), axis=-1, keepdims=True)
        mask1 = e_iota == i1
        probs_rest = jnp.where(mask1, -1.0, probs)
        v2 = jnp.max(probs_rest, axis=-1, keepdims=True)
        i2 = jnp.min(jnp.where(probs_rest == v2, e_iota, E), axis=-1,
                     keepdims=True)
        mask2 = e_iota == i2
        comb_ref[...] = (v1 * mask1.astype(jnp.float32)
                         + v2 * mask2.astype(jnp.float32))

    comb = comb_ref[...]
    ce = jnp.sum(jnp.where(
        jax.lax.broadcasted_iota(jnp.int32, comb.shape, 1) == e, comb, 0.0),
        axis=1, keepdims=True)
    contrib = ce * jnp.dot(
        x_ref[...], eW_ref[0], preferred_element_type=jnp.float32)

    @pl.when(e == 0)
    def _init():
        acc_ref[...] = contrib

    @pl.when(e > 0)
    def _accum():
        acc_ref[...] += contrib

    @pl.when(e == E - 1)
    def _classifier():
        ci = acc_ref[...] + jnp.dot(comb_ref[...], eb_ref[...],
                                    preferred_element_type=jnp.float32)
        ch = jnp.dot(ci, cW1_ref[...], preferred_element_type=jnp.float32)
        ch = ch + cb1_ref[0]
        mu = jnp.mean(ch, axis=-1, keepdims=True)
        var = jnp.mean((ch - mu) ** 2, axis=-1, keepdims=True)
        ch = ((ch - mu) * jax.lax.rsqrt(var + 1e-5) * cln_g_ref[0]
              + cln_b_ref[0])
        ch = jnp.maximum(ch, 0.0)
        logits = jnp.dot(ch, cW2_ref[...], preferred_element_type=jnp.float32)
        logits_ref[...] = logits + cb2_ref[0]


def kernel(x, gW1, gb1, gln_g, gln_b, gW2, gb2, eW, eb, cW1, cb1, cln_g,
           cln_b, cW2, cb2):
    B, H = x.shape
    E = eW.shape[0]
    C = cW2.shape[1]

    def row(v):
        return v.reshape(1, -1)

    full = lambda a: pl.BlockSpec(a.shape, lambda i: (0,) * a.ndim)
    out = pl.pallas_call(
        functools.partial(_fused_kernel, E=E),
        grid=(E,),
        in_specs=[
            full(x),
            full(gW1), full(row(gb1)), full(row(gln_g)), full(row(gln_b)),
            full(gW2), full(row(gb2)),
            pl.BlockSpec((1, H, H), lambda i: (i, 0, 0)),
            full(eb),
            full(cW1), full(row(cb1)), full(row(cln_g)), full(row(cln_b)),
            full(cW2), full(row(cb2)),
        ],
        out_specs=[
            pl.BlockSpec((B, C), lambda i: (0, 0)),
            pl.BlockSpec((B, E), lambda i: (0, 0)),
        ],
        out_shape=[
            jax.ShapeDtypeStruct((B, C), jnp.float32),
            jax.ShapeDtypeStruct((B, E), jnp.float32),
        ],
        scratch_shapes=[
            pltpu.VMEM((B, H), jnp.float32),
            pltpu.VMEM((B, E), jnp.float32),
        ],
    )(x, gW1, row(gb1), row(gln_g), row(gln_b), gW2, row(gb2), eW, eb,
      cW1, row(cb1), row(cln_g), row(cln_b), cW2, row(cb2))
    return out[0], out[1]
